# speedup vs baseline: 2.7206x; 2.7206x over previous
import jax
import jax.numpy as jnp
from jax.experimental import pallas as pl
from jax.experimental.pallas import tpu as pltpu

BT = 512
BI = 512


def _ffn_kernel(be_ref, x_ref, w1_ref, w3_ref, w2_ref, o_ref):
    io = pl.program_id(1)
    xb = x_ref[...]
    w1b = w1_ref[0]
    w3b = w3_ref[0]
    w2b = w2_ref[0]
    dn = (((1,), (1,)), ((), ()))
    h1 = jax.lax.dot_general(xb, w1b, dn, preferred_element_type=jnp.float32)
    h3 = jax.lax.dot_general(xb, w3b, dn, preferred_element_type=jnp.float32)
    h = h1 * jax.nn.sigmoid(h1) * h3
    contrib = jnp.dot(h, w2b, preferred_element_type=jnp.float32)

    @pl.when(io == 0)
    def _():
        o_ref[...] = contrib

    @pl.when(io > 0)
    def _():
        o_ref[...] += contrib


@jax.jit
def kernel(x, expert_indices, w1, w2, w3):
    T, D = x.shape
    A = expert_indices.shape[1]
    E, I, _ = w1.shape
    S = T * A
    NB = S // BT + E
    NI = I // BI

    e_flat = expert_indices.reshape(-1).astype(jnp.int32)
    order = jnp.argsort(e_flat).astype(jnp.int32)
    sorted_e = e_flat[order]
    counts = jnp.sum(
        (e_flat[None, :] == jnp.arange(E, dtype=jnp.int32)[:, None]).astype(jnp.int32),
        axis=1)
    blocks_per = (counts + BT - 1) // BT
    blocks_cum = jnp.cumsum(blocks_per)
    block_start = blocks_cum - blocks_per
    total_blocks = blocks_cum[-1]
    group_start = jnp.cumsum(counts) - counts

    j = jnp.arange(S, dtype=jnp.int32)
    dest = block_start[sorted_e] * BT + (j - group_start[sorted_e])

    b_ids = jnp.arange(NB, dtype=jnp.int32)
    be_raw = jnp.searchsorted(blocks_cum, b_ids, side="right").astype(jnp.int32)
    last_e = sorted_e[-1]
    block_expert = jnp.where(b_ids < total_blocks, jnp.minimum(be_raw, E - 1), last_e)

    tok_pad = jnp.zeros((NB * BT,), jnp.int32).at[dest].set(order // A)
    x_pad = x[tok_pad]

    grid_spec = pltpu.PrefetchScalarGridSpec(
        num_scalar_prefetch=1,
        grid=(NB, NI),
        in_specs=[
            pl.BlockSpec((BT, D), lambda b, io, be: (b, 0)),
            pl.BlockSpec((1, BI, D), lambda b, io, be: (be[b], io, 0)),
            pl.BlockSpec((1, BI, D), lambda b, io, be: (be[b], io, 0)),
            pl.BlockSpec((1, BI, D), lambda b, io, be: (be[b], io, 0)),
        ],
        out_specs=pl.BlockSpec((BT, D), lambda b, io, be: (b, 0)),
    )
    out_pad = pl.pallas_call(
        _ffn_kernel,
        grid_spec=grid_spec,
        out_shape=jax.ShapeDtypeStruct((NB * BT, D), jnp.float32),
        compiler_params=pltpu.CompilerParams(
            dimension_semantics=("parallel", "arbitrary"),
            vmem_limit_bytes=100 * 1024 * 1024,
        ),
    )(block_expert, x_pad, w1, w3, w2)

    row_of_pair = jnp.zeros((S,), jnp.int32).at[order].set(dest)
    out = out_pad[row_of_pair].reshape(T, A, D)
    return out


# speedup vs baseline: 3.6094x; 1.3267x over previous
import jax
import jax.numpy as jnp
from jax.experimental import pallas as pl
from jax.experimental.pallas import tpu as pltpu

BT = 512
BI = 512


def _ffn_kernel(be_ref, x_ref, w1_ref, w3_ref, w2_ref, o_ref):
    b = pl.program_id(0)
    io = pl.program_id(1)
    nb_active = be_ref[be_ref.shape[0] - 1]

    @pl.when(b < nb_active)
    def _():
        xb = x_ref[...].astype(jnp.bfloat16)
        w1b = w1_ref[0].astype(jnp.bfloat16)
        w3b = w3_ref[0].astype(jnp.bfloat16)
        w2b = w2_ref[0].astype(jnp.bfloat16)
        dn = (((1,), (1,)), ((), ()))
        h1 = jax.lax.dot_general(xb, w1b, dn, preferred_element_type=jnp.float32)
        h3 = jax.lax.dot_general(xb, w3b, dn, preferred_element_type=jnp.float32)
        h = (h1 * jax.nn.sigmoid(h1) * h3).astype(jnp.bfloat16)
        contrib = jnp.dot(h, w2b, preferred_element_type=jnp.float32)

        @pl.when(io == 0)
        def _():
            o_ref[...] = contrib

        @pl.when(io > 0)
        def _():
            o_ref[...] += contrib


@jax.jit
def kernel(x, expert_indices, w1, w2, w3):
    T, D = x.shape
    A = expert_indices.shape[1]
    E, I, _ = w1.shape
    S = T * A
    NB = S // BT + E
    NI = I // BI

    e_flat = expert_indices.reshape(-1).astype(jnp.int32)
    order = jnp.argsort(e_flat).astype(jnp.int32)
    sorted_e = e_flat[order]
    counts = jnp.sum(
        (e_flat[None, :] == jnp.arange(E, dtype=jnp.int32)[:, None]).astype(jnp.int32),
        axis=1)
    blocks_per = (counts + BT - 1) // BT
    blocks_cum = jnp.cumsum(blocks_per)
    block_start = blocks_cum - blocks_per
    total_blocks = blocks_cum[-1]
    group_start = jnp.cumsum(counts) - counts

    j = jnp.arange(S, dtype=jnp.int32)
    dest = block_start[sorted_e] * BT + (j - group_start[sorted_e])

    b_ids = jnp.arange(NB, dtype=jnp.int32)
    be_raw = jnp.searchsorted(blocks_cum, b_ids, side="right").astype(jnp.int32)
    last_e = sorted_e[-1]
    block_expert = jnp.where(b_ids < total_blocks, jnp.minimum(be_raw, E - 1), last_e)

    tok_pad = jnp.zeros((NB * BT,), jnp.int32).at[dest].set(order // A)
    x_pad = x[tok_pad]

    be_sched = jnp.concatenate([block_expert, total_blocks[None]])

    def w_map(b, io, be):
        nb = be[NB]
        return (be[jnp.minimum(b, nb - 1)], jnp.where(b < nb, io, NI - 1), 0)

    def row_map(b, io, be):
        return (jnp.minimum(b, be[NB] - 1), 0)

    grid_spec = pltpu.PrefetchScalarGridSpec(
        num_scalar_prefetch=1,
        grid=(NB, NI),
        in_specs=[
            pl.BlockSpec((BT, D), row_map),
            pl.BlockSpec((1, BI, D), w_map),
            pl.BlockSpec((1, BI, D), w_map),
            pl.BlockSpec((1, BI, D), w_map),
        ],
        out_specs=pl.BlockSpec((BT, D), row_map),
    )
    out_pad = pl.pallas_call(
        _ffn_kernel,
        grid_spec=grid_spec,
        out_shape=jax.ShapeDtypeStruct((NB * BT, D), jnp.float32),
        compiler_params=pltpu.CompilerParams(
            dimension_semantics=("parallel", "arbitrary"),
            vmem_limit_bytes=100 * 1024 * 1024,
        ),
    )(be_sched, x_pad, w1, w3, w2)

    row_of_pair = jnp.zeros((S,), jnp.int32).at[order].set(dest)
    out = out_pad[row_of_pair].reshape(T, A, D)
    return out
